# Initial kernel scaffold; baseline (speedup 1.0000x reference)
import jax
import jax.numpy as jnp
from jax.experimental import pallas as pl


def kernel(hidden_states, mask, enc_Wq, enc_bq, enc_Wk, enc_bk, dec_Wq, dec_bq, dec_Wk, dec_bk, residual_weights):
    raise NotImplementedError("write your pallas kernel here")


# speedup vs baseline: 450.9502x; 450.9502x over previous
import functools

import jax
import jax.numpy as jnp
from jax import lax
from jax.experimental import pallas as pl
from jax.experimental.pallas import tpu as pltpu
from jax.experimental.pallas import tpu_sc as plsc

_B, _L, _D = 4, 2048, 256
_T = 256
_NCH = _L // _T
_NC, _NS = 2, 16
_NW = _NC * _NS
_RPW = (_B * _L) // _NW


def _cumsum_col(x):
  n = x.shape[0]
  k = 1
  while k < n:
    x = x + jnp.concatenate([jnp.zeros((k, 1), x.dtype), x[:-k]], axis=0)
    k *= 2
  return x


def _routing_a(hn, wqt, bq, wkt, bk):
  qf = jnp.dot(hn, wqt, preferred_element_type=jnp.float32) + bq
  kf = jnp.dot(hn, wkt, preferred_element_type=jnp.float32) + bk
  qn = qf / jnp.maximum(
      jnp.sqrt(jnp.sum(qf * qf, axis=-1, keepdims=True)), 1e-12)
  kn = kf / jnp.maximum(
      jnp.sqrt(jnp.sum(kf * kf, axis=-1, keepdims=True)), 1e-12)
  cos = jnp.sum(qn[:-1] * kn[1:], axis=-1, keepdims=True)
  a_tail = jnp.clip(0.5 * (1.0 - cos), 0.0, 1.0)
  return jnp.concatenate(
      [jnp.ones((1, 1), jnp.float32), a_tail], axis=0)


def _enc_body(h_ref, mean_ref, m_ref, wqt_ref, bq_ref, wkt_ref, bk_ref,
              hn_ref, dest_ref, mnext_ref):
  b = pl.program_id(0)
  hn = h_ref[0] * lax.rsqrt(mean_ref[0] + 1e-6)
  hn_ref[0] = hn
  a = _routing_a(hn, wqt_ref[...], bq_ref[...], wkt_ref[...], bk_ref[...])
  bm = jnp.where((a > 0.5) & (m_ref[0] > 0.5), 1.0, 0.0)
  cs = _cumsum_col(bm)
  n = cs[_L - 1:_L, :]
  iota = lax.broadcasted_iota(jnp.int32, (_L, 1), 0).astype(jnp.float32)
  dest = jnp.where(bm > 0.5, cs - 1.0, n + iota - cs)
  dest_ref[0] = dest.astype(jnp.int32) + b * _L
  mnext_ref[0] = jnp.where(iota < n, 1.0, 0.0)


def _dec_body(h_ref, mean_ref, m_ref, enc_ref, wqt_ref, bq_ref, wkt_ref,
              bk_ref, z2_ref, p_ref, cb_ref):
  b = pl.program_id(0)
  hn = h_ref[0] * lax.rsqrt(mean_ref[0] + 1e-6)
  z2_ref[0, :, :_D] = hn
  z2_ref[0, :, _D:] = enc_ref[0]
  a = _routing_a(hn, wqt_ref[...], bq_ref[...], wkt_ref[...], bk_ref[...])
  bm = jnp.where((a > 0.5) & (m_ref[0] > 0.5), 1.0, 0.0)
  p_ref[0] = jnp.clip(a, 1e-4, 1.0 - 1e-4)
  cs = _cumsum_col(bm)
  cb = jnp.clip(cs - 1.0, 0.0, float(_L - 1))
  cb_ref[0] = cb.astype(jnp.int32) + b * _L


def _ema_exact_body(z2_ref, pr_ref, spc_ref, spr_ref, rw_ref, out_ref):
  jj = lax.broadcasted_iota(jnp.int32, (_T, _L), 1)
  ii0 = lax.broadcasted_iota(jnp.int32, (_T, _L), 0)

  def chunk(c, _):
    sp_col = spc_ref[0, pl.ds(c * _T, _T), :]
    expo = sp_col - spr_ref[0]
    causal = (ii0 + c * _T) >= jj
    w = pr_ref[0] * jnp.exp(jnp.where(causal, expo, -jnp.inf))
    o2 = jnp.dot(w, z2_ref[0], preferred_element_type=jnp.float32)
    out_ref[0, pl.ds(c * _T, _T), :] = (
        o2[:, :_D] + rw_ref[...] * o2[:, _D:])
    return 0

  lax.fori_loop(0, _NCH, chunk, 0)


def _ema_fast_body(z2_ref, pr_ref, spc_ref, spr_ref, rw_ref, out_ref):
  ii = lax.broadcasted_iota(jnp.int32, (_T, _T), 0)
  jj = lax.broadcasted_iota(jnp.int32, (_T, _T), 1)
  causal = ii >= jj

  def chunk(c, carry):
    y, sp_prev = carry
    sp_col = spc_ref[0, pl.ds(c * _T, _T), :]
    sp_row = spr_ref[0, :, pl.ds(c * _T, _T)]
    p_row = pr_ref[0, :, pl.ds(c * _T, _T)]
    zc = z2_ref[0, pl.ds(c * _T, _T), :]
    w = jnp.where(causal, p_row * jnp.exp(sp_col - sp_row), 0.0)
    o2 = jnp.dot(w, zc, preferred_element_type=jnp.float32)
    o2 = o2 + jnp.exp(sp_col - sp_prev) * y
    out_ref[0, pl.ds(c * _T, _T), :] = (
        o2[:, :_D] + rw_ref[...] * o2[:, _D:])
    return o2[_T - 1:_T, :], sp_col[_T - 1:_T, :]

  lax.fori_loop(0, _NCH, chunk,
                (jnp.zeros((1, 2 * _D), jnp.float32),
                 jnp.zeros((1, 1), jnp.float32)))


def _enc_call(h, mean, m, wqt, bq, wkt, bk):
  return pl.pallas_call(
      _enc_body,
      grid=(_B,),
      in_specs=[
          pl.BlockSpec((1, _L, _D), lambda b: (b, 0, 0)),
          pl.BlockSpec((1, _L, 1), lambda b: (b, 0, 0)),
          pl.BlockSpec((1, _L, 1), lambda b: (b, 0, 0)),
          pl.BlockSpec((_D, _D), lambda b: (0, 0)),
          pl.BlockSpec((1, _D), lambda b: (0, 0)),
          pl.BlockSpec((_D, _D), lambda b: (0, 0)),
          pl.BlockSpec((1, _D), lambda b: (0, 0)),
      ],
      out_specs=[
          pl.BlockSpec((1, _L, _D), lambda b: (b, 0, 0)),
          pl.BlockSpec((1, _L, 1), lambda b: (b, 0, 0)),
          pl.BlockSpec((1, _L, 1), lambda b: (b, 0, 0)),
      ],
      out_shape=[
          jax.ShapeDtypeStruct((_B, _L, _D), jnp.float32),
          jax.ShapeDtypeStruct((_B, _L, 1), jnp.int32),
          jax.ShapeDtypeStruct((_B, _L, 1), jnp.float32),
      ],
      compiler_params=pltpu.CompilerParams(
          dimension_semantics=("arbitrary",)),
  )(h, mean, m, wqt, bq, wkt, bk)


def _dec_call(h, mean, m, enc_n, wqt, bq, wkt, bk):
  return pl.pallas_call(
      _dec_body,
      grid=(_B,),
      in_specs=[
          pl.BlockSpec((1, _L, _D), lambda b: (b, 0, 0)),
          pl.BlockSpec((1, _L, 1), lambda b: (b, 0, 0)),
          pl.BlockSpec((1, _L, 1), lambda b: (b, 0, 0)),
          pl.BlockSpec((1, _L, _D), lambda b: (b, 0, 0)),
          pl.BlockSpec((_D, _D), lambda b: (0, 0)),
          pl.BlockSpec((1, _D), lambda b: (0, 0)),
          pl.BlockSpec((_D, _D), lambda b: (0, 0)),
          pl.BlockSpec((1, _D), lambda b: (0, 0)),
      ],
      out_specs=[
          pl.BlockSpec((1, _L, 2 * _D), lambda b: (b, 0, 0)),
          pl.BlockSpec((1, _L, 1), lambda b: (b, 0, 0)),
          pl.BlockSpec((1, _L, 1), lambda b: (b, 0, 0)),
      ],
      out_shape=[
          jax.ShapeDtypeStruct((_B, _L, 2 * _D), jnp.float32),
          jax.ShapeDtypeStruct((_B, _L, 1), jnp.float32),
          jax.ShapeDtypeStruct((_B, _L, 1), jnp.int32),
      ],
      compiler_params=pltpu.CompilerParams(
          dimension_semantics=("arbitrary",)),
  )(h, mean, m, enc_n, wqt, bq, wkt, bk)


def _ema_call(body, z2_exp, p_row, spad_col, spad_row, rw):
  return pl.pallas_call(
      body,
      grid=(_B,),
      in_specs=[
          pl.BlockSpec((1, _L, 2 * _D), lambda b: (b, 0, 0)),
          pl.BlockSpec((1, 1, _L), lambda b: (b, 0, 0)),
          pl.BlockSpec((1, _L, 1), lambda b: (b, 0, 0)),
          pl.BlockSpec((1, 1, _L), lambda b: (b, 0, 0)),
          pl.BlockSpec((1, 1), lambda b: (0, 0)),
      ],
      out_specs=pl.BlockSpec((1, _L, _D), lambda b: (b, 0, 0)),
      out_shape=jax.ShapeDtypeStruct((_B, _L, _D), jnp.float32),
      compiler_params=pltpu.CompilerParams(
          dimension_semantics=("arbitrary",)),
  )(z2_exp, p_row, spad_col, spad_row, rw)


@functools.cache
def _build_sc_kernels():
  mesh = plsc.VectorSubcoreMesh(
      core_axis_name="c", subcore_axis_name="s",
      num_cores=_NC, num_subcores=_NS)

  @functools.partial(
      pl.kernel,
      out_type=jax.ShapeDtypeStruct((_B * _L, _D), jnp.float32),
      mesh=mesh,
      scratch_types=[
          pltpu.VMEM((2, 128), jnp.int32),
          pltpu.VMEM((_RPW, _D), jnp.float32),
          pltpu.SemaphoreType.DMA,
      ],
  )
  def sc_permute(h_hbm, idx_hbm, out_hbm, idx_v, rows_v, sem):
    wid = lax.axis_index("s") * _NC + lax.axis_index("c")
    base = wid * _RPW
    pltpu.sync_copy(idx_hbm.at[wid], idx_v)
    pltpu.sync_copy(h_hbm.at[pl.ds(base, _RPW)], rows_v)
    for j in range(2):
      pltpu.async_copy(
          rows_v.at[pl.ds(j * 128, 128)], out_hbm.at[idx_v.at[j]], sem).wait()

  @functools.partial(
      pl.kernel,
      out_type=jax.ShapeDtypeStruct((_B * _L, 2 * _D), jnp.float32),
      mesh=mesh,
      scratch_types=[
          pltpu.VMEM((2, 128), jnp.int32),
          pltpu.VMEM((128, 2 * _D), jnp.float32),
          pltpu.SemaphoreType.DMA,
      ],
  )
  def sc_gather(z_hbm, idx_hbm, out_hbm, idx_v, rows_v, sem):
    wid = lax.axis_index("s") * _NC + lax.axis_index("c")
    base = wid * _RPW
    pltpu.sync_copy(idx_hbm.at[wid], idx_v)
    for j in range(2):
      pltpu.async_copy(z_hbm.at[idx_v.at[j]], rows_v, sem).wait()
      pltpu.sync_copy(rows_v, out_hbm.at[pl.ds(base + j * 128, 128)])

  return sc_permute, sc_gather


def _sc_permute(h_flat, idx):
  return _build_sc_kernels()[0](h_flat, idx)


def _sc_gather(z_flat, idx):
  return _build_sc_kernels()[1](z_flat, idx)


def kernel(hidden_states, mask, enc_Wq, enc_bq, enc_Wk, enc_bk,
           dec_Wq, dec_bq, dec_Wk, dec_bk, residual_weights):
  h = hidden_states
  m = mask.astype(jnp.float32).reshape(_B, _L, 1)

  enc_norms = []
  for i in range(2):
    mean = jnp.mean(h * h, axis=-1, keepdims=True)
    hn, dest, m_next = _enc_call(
        h, mean, m, enc_Wq[i].T, enc_bq[i].reshape(1, _D),
        enc_Wk[i].T, enc_bk[i].reshape(1, _D))
    enc_norms.append(hn)
    h = _sc_permute(
        hn.reshape(_B * _L, _D),
        dest.reshape(_NW, 2, 128)).reshape(_B, _L, _D)
    m = m_next

  enc_rev = enc_norms[::-1]
  for i in range(2):
    mean = jnp.mean(h * h, axis=-1, keepdims=True)
    z2, p, cb = _dec_call(
        h, mean, m, enc_rev[i], dec_Wq[i].T, dec_bq[i].reshape(1, _D),
        dec_Wk[i].T, dec_bk[i].reshape(1, _D))
    p2 = p.reshape(_B, _L)
    s = jnp.cumsum(jnp.log1p(-p2), axis=1)
    spad = jnp.concatenate([jnp.zeros((_B, 1), jnp.float32), s[:, :-1]],
                           axis=1)
    z2e = _sc_gather(
        z2.reshape(_B * _L, 2 * _D),
        cb.reshape(_NW, 2, 128)).reshape(_B, _L, 2 * _D)
    body = _ema_exact_body if i == 0 else _ema_fast_body
    h = _ema_call(body, z2e, p.reshape(_B, 1, _L), spad.reshape(_B, _L, 1),
                  spad.reshape(_B, 1, _L),
                  residual_weights[i].reshape(1, 1))

  return h


# speedup vs baseline: 466.1739x; 1.0338x over previous
import functools

import jax
import jax.numpy as jnp
from jax import lax
from jax.experimental import pallas as pl
from jax.experimental.pallas import tpu as pltpu
from jax.experimental.pallas import tpu_sc as plsc

_B, _L, _D = 4, 2048, 256
_T = 256
_NCH = _L // _T
_NC, _NS = 2, 16
_NW = _NC * _NS
_RPW = (_B * _L) // _NW


def _cumsum_col(x):
  n = x.shape[0]
  k = 1
  while k < n:
    x = x + jnp.concatenate([jnp.zeros((k, 1), x.dtype), x[:-k]], axis=0)
    k *= 2
  return x


def _routing_a(hn, wqt, bq, wkt, bk):
  qf = jnp.dot(hn, wqt, preferred_element_type=jnp.float32) + bq
  kf = jnp.dot(hn, wkt, preferred_element_type=jnp.float32) + bk
  qn = qf / jnp.maximum(
      jnp.sqrt(jnp.sum(qf * qf, axis=-1, keepdims=True)), 1e-12)
  kn = kf / jnp.maximum(
      jnp.sqrt(jnp.sum(kf * kf, axis=-1, keepdims=True)), 1e-12)
  cos = jnp.sum(qn[:-1] * kn[1:], axis=-1, keepdims=True)
  a_tail = jnp.clip(0.5 * (1.0 - cos), 0.0, 1.0)
  return jnp.concatenate(
      [jnp.ones((1, 1), jnp.float32), a_tail], axis=0)


def _enc_body(h_ref, mean_ref, m_ref, wqt_ref, bq_ref, wkt_ref, bk_ref,
              hn_ref, dest_ref, mnext_ref):
  b = pl.program_id(0)
  hn = h_ref[0] * lax.rsqrt(mean_ref[0] + 1e-6)
  hn_ref[0] = hn
  a = _routing_a(hn, wqt_ref[...], bq_ref[...], wkt_ref[...], bk_ref[...])
  bm = jnp.where((a > 0.5) & (m_ref[0] > 0.5), 1.0, 0.0)
  cs = _cumsum_col(bm)
  n = cs[_L - 1:_L, :]
  iota = lax.broadcasted_iota(jnp.int32, (_L, 1), 0).astype(jnp.float32)
  dest = jnp.where(bm > 0.5, cs - 1.0, n + iota - cs)
  dest_ref[0] = dest.astype(jnp.int32) + b * _L
  mnext_ref[0] = jnp.where(iota < n, 1.0, 0.0)


def _dec_body(h_ref, mean_ref, m_ref, enc_ref, wqt_ref, bq_ref, wkt_ref,
              bk_ref, z2_ref, p_ref, cb_ref):
  b = pl.program_id(0)
  hn = h_ref[0] * lax.rsqrt(mean_ref[0] + 1e-6)
  z2_ref[0, :, :_D] = hn
  z2_ref[0, :, _D:] = enc_ref[0]
  a = _routing_a(hn, wqt_ref[...], bq_ref[...], wkt_ref[...], bk_ref[...])
  bm = jnp.where((a > 0.5) & (m_ref[0] > 0.5), 1.0, 0.0)
  p_ref[0] = jnp.clip(a, 1e-4, 1.0 - 1e-4)
  cs = _cumsum_col(bm)
  cb = jnp.clip(cs - 1.0, 0.0, float(_L - 1))
  cb_ref[0] = cb.astype(jnp.int32) + b * _L


def _ema_exact_body(z2_ref, pr_ref, spc_ref, spr_ref, rw_ref, out_ref):
  for c in range(_NCH):
    kk = (c + 1) * _T
    sp_col = spc_ref[0, c * _T:(c + 1) * _T, :]
    expo = sp_col - spr_ref[0, :, :kk]
    ii = lax.broadcasted_iota(jnp.int32, (_T, kk), 0) + c * _T
    jj = lax.broadcasted_iota(jnp.int32, (_T, kk), 1)
    w = pr_ref[0, :, :kk] * jnp.exp(jnp.where(ii >= jj, expo, -jnp.inf))
    o2 = jnp.dot(w, z2_ref[0, :kk, :], preferred_element_type=jnp.float32)
    out_ref[0, c * _T:(c + 1) * _T, :] = (
        o2[:, :_D] + rw_ref[...] * o2[:, _D:])


def _ema_fast_body(z2_ref, pr_ref, spc_ref, spr_ref, rw_ref, out_ref):
  ii = lax.broadcasted_iota(jnp.int32, (_T, _T), 0)
  jj = lax.broadcasted_iota(jnp.int32, (_T, _T), 1)
  causal = ii >= jj

  def chunk(c, carry):
    y, sp_prev = carry
    sp_col = spc_ref[0, pl.ds(c * _T, _T), :]
    sp_row = spr_ref[0, :, pl.ds(c * _T, _T)]
    p_row = pr_ref[0, :, pl.ds(c * _T, _T)]
    zc = z2_ref[0, pl.ds(c * _T, _T), :]
    w = jnp.where(causal, p_row * jnp.exp(sp_col - sp_row), 0.0)
    o2 = jnp.dot(w, zc, preferred_element_type=jnp.float32)
    o2 = o2 + jnp.exp(sp_col - sp_prev) * y
    out_ref[0, pl.ds(c * _T, _T), :] = (
        o2[:, :_D] + rw_ref[...] * o2[:, _D:])
    return o2[_T - 1:_T, :], sp_col[_T - 1:_T, :]

  lax.fori_loop(0, _NCH, chunk,
                (jnp.zeros((1, 2 * _D), jnp.float32),
                 jnp.zeros((1, 1), jnp.float32)))


def _enc_call(h, mean, m, wqt, bq, wkt, bk):
  return pl.pallas_call(
      _enc_body,
      grid=(_B,),
      in_specs=[
          pl.BlockSpec((1, _L, _D), lambda b: (b, 0, 0)),
          pl.BlockSpec((1, _L, 1), lambda b: (b, 0, 0)),
          pl.BlockSpec((1, _L, 1), lambda b: (b, 0, 0)),
          pl.BlockSpec((_D, _D), lambda b: (0, 0)),
          pl.BlockSpec((1, _D), lambda b: (0, 0)),
          pl.BlockSpec((_D, _D), lambda b: (0, 0)),
          pl.BlockSpec((1, _D), lambda b: (0, 0)),
      ],
      out_specs=[
          pl.BlockSpec((1, _L, _D), lambda b: (b, 0, 0)),
          pl.BlockSpec((1, _L, 1), lambda b: (b, 0, 0)),
          pl.BlockSpec((1, _L, 1), lambda b: (b, 0, 0)),
      ],
      out_shape=[
          jax.ShapeDtypeStruct((_B, _L, _D), jnp.float32),
          jax.ShapeDtypeStruct((_B, _L, 1), jnp.int32),
          jax.ShapeDtypeStruct((_B, _L, 1), jnp.float32),
      ],
      compiler_params=pltpu.CompilerParams(
          dimension_semantics=("arbitrary",)),
  )(h, mean, m, wqt, bq, wkt, bk)


def _dec_call(h, mean, m, enc_n, wqt, bq, wkt, bk):
  return pl.pallas_call(
      _dec_body,
      grid=(_B,),
      in_specs=[
          pl.BlockSpec((1, _L, _D), lambda b: (b, 0, 0)),
          pl.BlockSpec((1, _L, 1), lambda b: (b, 0, 0)),
          pl.BlockSpec((1, _L, 1), lambda b: (b, 0, 0)),
          pl.BlockSpec((1, _L, _D), lambda b: (b, 0, 0)),
          pl.BlockSpec((_D, _D), lambda b: (0, 0)),
          pl.BlockSpec((1, _D), lambda b: (0, 0)),
          pl.BlockSpec((_D, _D), lambda b: (0, 0)),
          pl.BlockSpec((1, _D), lambda b: (0, 0)),
      ],
      out_specs=[
          pl.BlockSpec((1, _L, 2 * _D), lambda b: (b, 0, 0)),
          pl.BlockSpec((1, _L, 1), lambda b: (b, 0, 0)),
          pl.BlockSpec((1, _L, 1), lambda b: (b, 0, 0)),
      ],
      out_shape=[
          jax.ShapeDtypeStruct((_B, _L, 2 * _D), jnp.float32),
          jax.ShapeDtypeStruct((_B, _L, 1), jnp.float32),
          jax.ShapeDtypeStruct((_B, _L, 1), jnp.int32),
      ],
      compiler_params=pltpu.CompilerParams(
          dimension_semantics=("arbitrary",)),
  )(h, mean, m, enc_n, wqt, bq, wkt, bk)


def _ema_call(body, z2_exp, p_row, spad_col, spad_row, rw):
  return pl.pallas_call(
      body,
      grid=(_B,),
      in_specs=[
          pl.BlockSpec((1, _L, 2 * _D), lambda b: (b, 0, 0)),
          pl.BlockSpec((1, 1, _L), lambda b: (b, 0, 0)),
          pl.BlockSpec((1, _L, 1), lambda b: (b, 0, 0)),
          pl.BlockSpec((1, 1, _L), lambda b: (b, 0, 0)),
          pl.BlockSpec((1, 1), lambda b: (0, 0)),
      ],
      out_specs=pl.BlockSpec((1, _L, _D), lambda b: (b, 0, 0)),
      out_shape=jax.ShapeDtypeStruct((_B, _L, _D), jnp.float32),
      compiler_params=pltpu.CompilerParams(
          dimension_semantics=("arbitrary",)),
  )(z2_exp, p_row, spad_col, spad_row, rw)


@functools.cache
def _build_sc_kernels():
  mesh = plsc.VectorSubcoreMesh(
      core_axis_name="c", subcore_axis_name="s",
      num_cores=_NC, num_subcores=_NS)

  @functools.partial(
      pl.kernel,
      out_type=jax.ShapeDtypeStruct((_B * _L, _D), jnp.float32),
      mesh=mesh,
      scratch_types=[
          pltpu.VMEM((2, 128), jnp.int32),
          pltpu.VMEM((_RPW, _D), jnp.float32),
          pltpu.SemaphoreType.DMA,
      ],
  )
  def sc_permute(h_hbm, idx_hbm, out_hbm, idx_v, rows_v, sem):
    wid = lax.axis_index("s") * _NC + lax.axis_index("c")
    base = wid * _RPW
    pltpu.sync_copy(idx_hbm.at[wid], idx_v)
    pltpu.sync_copy(h_hbm.at[pl.ds(base, _RPW)], rows_v)
    for j in range(2):
      pltpu.async_copy(
          rows_v.at[pl.ds(j * 128, 128)], out_hbm.at[idx_v.at[j]], sem).wait()

  _GC = 64
  _NG = _RPW // _GC
  _NBUF = 3

  @functools.partial(
      pl.kernel,
      out_type=jax.ShapeDtypeStruct((_B * _L, 2 * _D), jnp.float32),
      mesh=mesh,
      scratch_types=[
          pltpu.VMEM((_NG, _GC), jnp.int32),
          [pltpu.VMEM((_GC, 2 * _D), jnp.float32) for _ in range(_NBUF)],
          [pltpu.SemaphoreType.DMA for _ in range(_NBUF)],
          [pltpu.SemaphoreType.DMA for _ in range(_NBUF)],
      ],
  )
  def sc_gather(z_hbm, idx_hbm, out_hbm, idx_v, bufs, gsems, osems):
    wid = lax.axis_index("s") * _NC + lax.axis_index("c")
    base = wid * _RPW
    pltpu.sync_copy(idx_hbm.at[wid], idx_v)
    pend_out = {}
    prev = None
    for k in range(_NG):
      b = k % _NBUF
      if b in pend_out:
        pend_out.pop(b).wait()
      gd = pltpu.async_copy(z_hbm.at[idx_v.at[k]], bufs[b], gsems[b])
      if prev is not None:
        pb, pgd, pk = prev
        pgd.wait()
        pend_out[pb] = pltpu.async_copy(
            bufs[pb], out_hbm.at[pl.ds(base + pk * _GC, _GC)], osems[pb])
      prev = (b, gd, k)
    pb, pgd, pk = prev
    pgd.wait()
    pend_out[pb] = pltpu.async_copy(
        bufs[pb], out_hbm.at[pl.ds(base + pk * _GC, _GC)], osems[pb])
    for d in pend_out.values():
      d.wait()

  return sc_permute, sc_gather


def _sc_permute(h_flat, idx):
  return _build_sc_kernels()[0](h_flat, idx)


def _sc_gather(z_flat, idx):
  return _build_sc_kernels()[1](z_flat, idx)


def kernel(hidden_states, mask, enc_Wq, enc_bq, enc_Wk, enc_bk,
           dec_Wq, dec_bq, dec_Wk, dec_bk, residual_weights):
  h = hidden_states
  m = mask.astype(jnp.float32).reshape(_B, _L, 1)

  enc_norms = []
  for i in range(2):
    mean = jnp.mean(h * h, axis=-1, keepdims=True)
    hn, dest, m_next = _enc_call(
        h, mean, m, enc_Wq[i].T, enc_bq[i].reshape(1, _D),
        enc_Wk[i].T, enc_bk[i].reshape(1, _D))
    enc_norms.append(hn)
    h = _sc_permute(
        hn.reshape(_B * _L, _D),
        dest.reshape(_NW, 2, 128)).reshape(_B, _L, _D)
    m = m_next

  enc_rev = enc_norms[::-1]
  for i in range(2):
    mean = jnp.mean(h * h, axis=-1, keepdims=True)
    z2, p, cb = _dec_call(
        h, mean, m, enc_rev[i], dec_Wq[i].T, dec_bq[i].reshape(1, _D),
        dec_Wk[i].T, dec_bk[i].reshape(1, _D))
    p2 = p.reshape(_B, _L)
    s = jnp.cumsum(jnp.log1p(-p2), axis=1)
    spad = jnp.concatenate([jnp.zeros((_B, 1), jnp.float32), s[:, :-1]],
                           axis=1)
    z2e = _sc_gather(
        z2.reshape(_B * _L, 2 * _D),
        cb.reshape(_NW, 4, 64)).reshape(_B, _L, 2 * _D)
    body = _ema_exact_body if i == 0 else _ema_fast_body
    h = _ema_call(body, z2e, p.reshape(_B, 1, _L), spad.reshape(_B, _L, 1),
                  spad.reshape(_B, 1, _L),
                  residual_weights[i].reshape(1, 1))

  return h
